# baseline (speedup 1.0000x reference)
import functools

import jax
import jax.numpy as jnp
from jax import lax
from jax.experimental import pallas as pl
from jax.experimental.pallas import tpu as pltpu
from jax.experimental.pallas import tpu_sc as plsc

_NUM_EXPERTS = 64
_TOP_K = 8
_N_GROUP = 16
_TOPK_GROUP = 4
_EPG = 4
_SCALE = 2.5
_NEG_INF = float("-inf")

_NC, _NS, _L = 2, 16, 16
_NW = _NC * _NS
_N_TOK = 8192
_TPW = _N_TOK // _NW
_TILES = _TPW // _L


def _scores_body(x_ref, wt_ref, o_ref):
    logits = jnp.dot(x_ref[...], wt_ref[...],
                     preferred_element_type=jnp.float32)
    o_ref[...] = jax.nn.sigmoid(logits)


def _tc_scores(x, wt, tb=512):
    n_tok, h = x.shape
    return pl.pallas_call(
        _scores_body,
        grid=(n_tok // tb,),
        in_specs=[pl.BlockSpec((tb, h), lambda i: (i, 0)),
                  pl.BlockSpec((h, _NUM_EXPERTS), lambda i: (0, 0))],
        out_specs=pl.BlockSpec((tb, _NUM_EXPERTS), lambda i: (i, 0)),
        out_shape=jax.ShapeDtypeStruct((n_tok, _NUM_EXPERTS), jnp.float32),
        compiler_params=pltpu.CompilerParams(
            dimension_semantics=("arbitrary",),
        ),
    )(x, wt)


def _splat_i(v):
    return jnp.full((_L,), v, dtype=jnp.int32)


def _splat_f(v):
    return jnp.full((_L,), v, dtype=jnp.float32)


def _sc_route_body(s_hbm, idx_hbm, w_hbm, s_v, idx_v, w_v, sem):
    wid = lax.axis_index("s") * _NC + lax.axis_index("c")
    base = wid * _TPW
    pltpu.sync_copy(s_hbm.at[pl.ds(base * _NUM_EXPERTS, _TPW * _NUM_EXPERTS)],
                    s_v)

    lane = lax.broadcasted_iota(jnp.int32, (_L,), 0)

    def tile_body(ti, carry):
        row0 = (ti * _L + lane) * _NUM_EXPERTS

        gs = []
        for g in range(_N_GROUP):
            m = [plsc.load_gather(s_v, [row0 + (_EPG * g + j)])
                 for j in range(_EPG)]
            t2 = jnp.maximum(m[0] + m[1], m[0] + m[2])
            t2 = jnp.maximum(t2, m[0] + m[3])
            t2 = jnp.maximum(t2, m[1] + m[2])
            t2 = jnp.maximum(t2, m[1] + m[3])
            gs.append(jnp.maximum(t2, m[2] + m[3]))

        neg = _splat_f(_NEG_INF)
        gsel = []
        for _ in range(_TOPK_GROUP):
            mx = gs[0]
            for g in range(1, _N_GROUP):
                mx = jnp.maximum(mx, gs[g])
            gid = _splat_i(_N_GROUP)
            ngs = []
            for g in range(_N_GROUP):
                eq = gs[g] == mx
                gid = jnp.minimum(gid, jnp.where(eq, _splat_i(g), gid))
                ngs.append(jnp.where(eq, neg, gs[g]))
            gs = ngs
            gsel.append(gid)

        eids = []
        vals = []
        for i in range(_TOPK_GROUP):
            for j in range(_EPG):
                eid = gsel[i] * _EPG + j
                eids.append(eid)
                vals.append(plsc.load_gather(s_v, [row0 + eid]))

        idx_cols = []
        w_cols = []
        for _ in range(_TOP_K):
            mx = vals[0]
            for c in range(1, 16):
                mx = jnp.maximum(mx, vals[c])
            first = _splat_i(_NUM_EXPERTS)
            nvals = []
            for c in range(16):
                eq = vals[c] == mx
                first = jnp.minimum(first, jnp.where(eq, eids[c], first))
                nvals.append(jnp.where(eq, neg, vals[c]))
            vals = nvals
            idx_cols.append(first)
            w_cols.append(mx)

        denom = w_cols[0]
        for r in range(1, _TOP_K):
            denom = denom + w_cols[r]
        scale = _SCALE / (denom + 1e-20)
        out0 = (ti * _L + lane) * _TOP_K
        for r in range(_TOP_K):
            plsc.store_scatter(idx_v, [out0 + r], idx_cols[r])
            plsc.store_scatter(w_v, [out0 + r], w_cols[r] * scale)
        return carry

    lax.fori_loop(0, _TILES, tile_body, 0)

    pltpu.sync_copy(idx_v, idx_hbm.at[pl.ds(base * _TOP_K, _TPW * _TOP_K)])
    pltpu.sync_copy(w_v, w_hbm.at[pl.ds(base * _TOP_K, _TPW * _TOP_K)])


def _sc_route(scores):
    mesh = plsc.VectorSubcoreMesh(core_axis_name="c", subcore_axis_name="s")
    f = functools.partial(
        pl.kernel,
        mesh=mesh,
        out_type=(jax.ShapeDtypeStruct((_N_TOK * _TOP_K,), jnp.int32),
                  jax.ShapeDtypeStruct((_N_TOK * _TOP_K,), jnp.float32)),
        scratch_types=[
            pltpu.VMEM((_TPW * _NUM_EXPERTS,), jnp.float32),
            pltpu.VMEM((_TPW * _TOP_K,), jnp.int32),
            pltpu.VMEM((_TPW * _TOP_K,), jnp.float32),
            pltpu.SemaphoreType.DMA,
        ],
        compiler_params=pltpu.CompilerParams(needs_layout_passes=False),
    )(_sc_route_body)
    idx_flat, w_flat = f(scores.reshape(-1))
    return (idx_flat.reshape(_N_TOK, _TOP_K),
            w_flat.reshape(_N_TOK, _TOP_K))


@jax.jit
def kernel(hidden_states, weight, e_score_correction_bias):
    bsz, seq_len, h = hidden_states.shape
    n_tok = bsz * seq_len
    x = hidden_states.reshape(n_tok, h)
    wt = weight.astype(jnp.float32).T
    scores = _tc_scores(x, wt)
    return _sc_route(scores)
